# Initial kernel scaffold; baseline (speedup 1.0000x reference)
import jax
import jax.numpy as jnp
from jax.experimental import pallas as pl


def kernel(x, edge_index, batch, params):
    raise NotImplementedError("write your pallas kernel here")


# speedup vs baseline: 3.8899x; 3.8899x over previous
import functools
import math

import jax
import jax.numpy as jnp
from jax import lax
from jax.experimental import pallas as pl
from jax.experimental.pallas import tpu as pltpu
from jax.experimental.pallas import tpu_sc as plsc


def _make_sc_scatter(n_nodes, n_edges, feat):
    nc, ns = 2, 16
    nw = nc * ns
    chunk = 128
    n_chunks = -(-n_edges // (nw * chunk))
    epw = n_chunks * chunk
    arows = (n_nodes + 16) // 16 * 16
    zrows = -(-arows // (ns * 8)) * 8
    zlast = arows - zrows * (ns - 1)
    orows = -(-n_nodes // (ns * 8)) * 8
    olast = n_nodes - orows * (ns - 1)
    assert 0 < zlast <= zrows and 0 < olast <= orows
    mesh = plsc.VectorSubcoreMesh(core_axis_name="c", subcore_axis_name="s",
                                  num_cores=nc, num_subcores=ns)

    @functools.partial(
        pl.kernel,
        mesh=mesh,
        out_type=jax.ShapeDtypeStruct((nc, n_nodes, feat), jnp.float32),
        scratch_types=[
            pltpu.VMEM((n_chunks, chunk), jnp.int32),
            pltpu.VMEM((n_chunks, chunk), jnp.int32),
            pltpu.VMEM((chunk, feat), jnp.float32),
            pltpu.VMEM_SHARED((arows, feat), jnp.float32),
            pltpu.SemaphoreType.DMA,
        ],
    )
    def sc_scatter(h_hbm, src_hbm, dst_hbm, zeros_hbm, out_hbm,
                   src_v, dst_v, rows_v, agg_sh, sem):
        cid = lax.axis_index("c")
        sid = lax.axis_index("s")
        wid = cid * ns + sid
        zoff = pl.multiple_of(sid * zrows, 8)
        ooff = pl.multiple_of(sid * orows, 8)
        @pl.when(sid < ns - 1)
        def _():
            pltpu.sync_copy(zeros_hbm, agg_sh.at[pl.ds(zoff, zrows)])

        @pl.when(sid == ns - 1)
        def _():
            pltpu.sync_copy(zeros_hbm.at[pl.ds(0, zlast)],
                            agg_sh.at[pl.ds((ns - 1) * zrows, zlast)])

        pltpu.sync_copy(src_hbm.at[wid], src_v)
        pltpu.sync_copy(dst_hbm.at[wid], dst_v)
        plsc.subcore_barrier()

        def body(j, carry):
            pltpu.async_copy(h_hbm.at[src_v.at[j]], rows_v, sem).wait()
            pltpu.sync_copy(rows_v, agg_sh.at[dst_v.at[j]], add=True)
            return carry

        lax.fori_loop(0, n_chunks, body, 0, unroll=False)
        plsc.subcore_barrier()

        @pl.when(sid < ns - 1)
        def _():
            pltpu.sync_copy(agg_sh.at[pl.ds(ooff, orows)],
                            out_hbm.at[cid, pl.ds(ooff, orows)])

        @pl.when(sid == ns - 1)
        def _():
            pltpu.sync_copy(agg_sh.at[pl.ds((ns - 1) * orows, olast)],
                            out_hbm.at[cid, pl.ds((ns - 1) * orows, olast)])

    return sc_scatter, nw, epw, zrows


def _tc_encoder(x, w, b, rblk):
    n, d = x.shape
    h = w.shape[1]

    def body(x_ref, w_ref, b_ref, o_ref):
        o_ref[...] = (
            jnp.dot(x_ref[...], w_ref[...], preferred_element_type=jnp.float32, precision=lax.Precision.HIGHEST)
            + b_ref[...]
        )

    return pl.pallas_call(
        body,
        grid=(n // rblk,),
        in_specs=[
            pl.BlockSpec((rblk, d), lambda i: (i, 0)),
            pl.BlockSpec((d, h), lambda i: (0, 0)),
            pl.BlockSpec((1, h), lambda i: (0, 0)),
        ],
        out_specs=pl.BlockSpec((rblk, h), lambda i: (i, 0)),
        out_shape=jax.ShapeDtypeStruct((n, h), jnp.float32),
        compiler_params=pltpu.CompilerParams(
            dimension_semantics=("parallel",)),
    )(x, w, b)


def _tc_layer(h, agg2, epsp, w1, b1, w2, b2, rblk):
    n, d = h.shape

    def body(eps_ref, h_ref, a_ref, w1_ref, b1_ref, w2_ref, b2_ref, o_ref):
        z = h_ref[...] * eps_ref[0, 0] + a_ref[0] + a_ref[1]
        t = jnp.dot(z, w1_ref[...], preferred_element_type=jnp.float32, precision=lax.Precision.HIGHEST) + b1_ref[...]
        t = jnp.maximum(t, 0.0)
        u = jnp.dot(t, w2_ref[...], preferred_element_type=jnp.float32, precision=lax.Precision.HIGHEST) + b2_ref[...]
        o_ref[...] = jnp.maximum(u, 0.0)

    return pl.pallas_call(
        body,
        grid=(n // rblk,),
        in_specs=[
            pl.BlockSpec(memory_space=pltpu.SMEM),
            pl.BlockSpec((rblk, d), lambda i: (i, 0)),
            pl.BlockSpec((2, rblk, d), lambda i: (0, i, 0)),
            pl.BlockSpec((d, d), lambda i: (0, 0)),
            pl.BlockSpec((1, d), lambda i: (0, 0)),
            pl.BlockSpec((d, d), lambda i: (0, 0)),
            pl.BlockSpec((1, d), lambda i: (0, 0)),
        ],
        out_specs=pl.BlockSpec((rblk, d), lambda i: (i, 0)),
        out_shape=jax.ShapeDtypeStruct((n, d), jnp.float32),
        compiler_params=pltpu.CompilerParams(
            dimension_semantics=("parallel",)),
    )(epsp, h, agg2, w1, b1, w2, b2)


def _tc_pool(h, batch_col, n_graphs, w1, b1, w2, b2, rblk):
    n, d = h.shape
    o = w2.shape[1]
    nsteps = n // rblk

    def body(h_ref, b_ref, w1_ref, b1_ref, w2_ref, b2_ref, o_ref,
             summ_s, cnt_s):
        i = pl.program_id(0)

        @pl.when(i == 0)
        def _():
            summ_s[...] = jnp.zeros_like(summ_s)
            cnt_s[...] = jnp.zeros_like(cnt_s)

        onehot = (b_ref[...] == lax.broadcasted_iota(
            jnp.int32, (rblk, n_graphs), 1)).astype(jnp.float32)
        summ_s[...] += lax.dot_general(
            onehot, h_ref[...], (((0,), (0,)), ((), ())),
            preferred_element_type=jnp.float32, precision=lax.Precision.HIGHEST)
        cnt_s[...] += jnp.sum(onehot, axis=0, keepdims=True)

        @pl.when(i == nsteps - 1)
        def _():
            inv = 1.0 / jnp.maximum(cnt_s[...], 1.0)
            eye = (lax.broadcasted_iota(jnp.int32, (n_graphs, n_graphs), 0)
                   == lax.broadcasted_iota(jnp.int32, (n_graphs, n_graphs), 1))
            diagm = jnp.where(eye, inv, 0.0)
            pooled = jnp.dot(diagm, summ_s[...],
                             preferred_element_type=jnp.float32, precision=lax.Precision.HIGHEST)
            t = jnp.dot(pooled, w1_ref[...],
                        preferred_element_type=jnp.float32, precision=lax.Precision.HIGHEST) + b1_ref[...]
            t = jnp.maximum(t, 0.0)
            o_ref[...] = jnp.dot(t, w2_ref[...],
                                 preferred_element_type=jnp.float32, precision=lax.Precision.HIGHEST) + b2_ref[...]

    return pl.pallas_call(
        body,
        grid=(nsteps,),
        in_specs=[
            pl.BlockSpec((rblk, d), lambda i: (i, 0)),
            pl.BlockSpec((rblk, 1), lambda i: (i, 0)),
            pl.BlockSpec((d, d), lambda i: (0, 0)),
            pl.BlockSpec((1, d), lambda i: (0, 0)),
            pl.BlockSpec((d, o), lambda i: (0, 0)),
            pl.BlockSpec((1, o), lambda i: (0, 0)),
        ],
        out_specs=pl.BlockSpec((n_graphs, o), lambda i: (0, 0)),
        out_shape=jax.ShapeDtypeStruct((n_graphs, o), jnp.float32),
        scratch_shapes=[
            pltpu.VMEM((n_graphs, d), jnp.float32),
            pltpu.VMEM((1, n_graphs), jnp.float32),
        ],
        compiler_params=pltpu.CompilerParams(
            dimension_semantics=("arbitrary",)),
    )(h, batch_col, w1, b1, w2, b2)


_BN_SCALE = 1.0 / math.sqrt(1.0 + 1e-5)


def _fold_mlp(p, post_g=None, post_b=None):
    s1 = p["g"] * _BN_SCALE
    w1 = p["w1"] * s1[None, :]
    b1 = p["b1"] * s1 + p["bb"]
    if post_g is not None:
        s2 = post_g * _BN_SCALE
        w2 = p["w2"] * s2[None, :]
        b2 = p["b2"] * s2 + post_b
    else:
        w2, b2 = p["w2"], p["b2"]
    return w1, b1[None, :], w2, b2[None, :]


def kernel(x, edge_index, batch, params):
    n, d = x.shape
    e = edge_index.shape[1]
    n_graphs = 128
    n_layers = 3
    rblk = 1000

    sc_scatter, nw, epw, zrows = _make_sc_scatter(n, e, d)
    pad = nw * epw - e
    src = edge_index[0].astype(jnp.int32)
    dst = edge_index[1].astype(jnp.int32)
    src_p = jnp.concatenate([src, jnp.zeros((pad,), jnp.int32)]).reshape(
        nw, epw // 128, 128)
    dst_p = jnp.concatenate([dst, jnp.full((pad,), n, jnp.int32)]).reshape(
        nw, epw // 128, 128)
    zeros_hbm = jnp.zeros((zrows, d), jnp.float32)
    batch_col = batch.astype(jnp.int32).reshape(n, 1)

    h = _tc_encoder(x, params["enc_W"], params["enc_b"][None, :], rblk)
    for i in range(n_layers):
        p = params["conv%d" % i]
        agg2 = sc_scatter(h, src_p, dst_p, zeros_hbm)
        w1, b1, w2, b2 = _fold_mlp(
            p, params["bn%d_g" % i], params["bn%d_b" % i])
        epsp = (1.0 + p["eps"]).astype(jnp.float32).reshape(1, 1)
        h = _tc_layer(h, agg2, epsp, w1, b1, w2, b2, rblk)

    w1, b1, w2, b2 = _fold_mlp(params["mlp"])
    return _tc_pool(h, batch_col, n_graphs, w1, b1, w2, b2, rblk)
